# baseline (speedup 1.0000x reference)
import jax
import jax.numpy as jnp
from jax import lax
from jax.experimental import pallas as pl
from jax.experimental.pallas import tpu as pltpu
from jax.experimental.pallas import tpu_sc as plsc

BATCH = 4096
SEQLEN = 200
EMBED = 64

NC = 2
NS = 16
NW = NC * NS
ROWS_PER_W = BATCH // NW
RB = 1
ITERS = ROWS_PER_W // (2 * RB)
TAIL_L0 = SEQLEN - 16


def _sc_body(x_hbm, pos_hbm, out_hbm, x_v, pos_v, buf0, buf1, sem0, sem1):
    wid = lax.axis_index("s") * NC + lax.axis_index("c")
    base = wid * ROWS_PER_W

    pltpu.sync_copy(pos_hbm, pos_v)
    pltpu.sync_copy(x_hbm.at[pl.ds(base, ROWS_PER_W)], x_v)

    def emit16(buf, rr, l0, xv):
        for j in range(16):
            l = l0 + j
            s = xv[j]
            for dd in range(4):
                sl = pl.ds(dd * 16, 16)
                buf[rr, l, sl] = pos_v[l, sl] + s

    def compute_pair(buf, r0):
        def chunk(lc, carry):
            l0 = pl.multiple_of(lc * 16, 16)
            for rr in range(RB):
                xv = x_v[r0 + rr, pl.ds(l0, 16)]
                emit16(buf, rr, l0, xv)
            return carry

        lax.fori_loop(0, SEQLEN // 16, chunk, 0)
        for rr in range(RB):
            xv = x_v[r0 + rr, pl.ds(TAIL_L0, 16)]
            emit16(buf, rr, TAIL_L0, xv)

    def body(i, carry):
        r0 = 2 * RB * i

        @pl.when(i > 0)
        def _():
            pltpu.make_async_copy(buf0, out_hbm.at[pl.ds(0, RB)], sem0).wait()

        compute_pair(buf0, r0)
        pltpu.make_async_copy(buf0, out_hbm.at[pl.ds(base + r0, RB)], sem0).start()

        @pl.when(i > 0)
        def _():
            pltpu.make_async_copy(buf1, out_hbm.at[pl.ds(0, RB)], sem1).wait()

        compute_pair(buf1, r0 + RB)
        pltpu.make_async_copy(buf1, out_hbm.at[pl.ds(base + r0 + RB, RB)], sem1).start()
        return carry

    lax.fori_loop(0, ITERS, body, 0)
    pltpu.make_async_copy(buf0, out_hbm.at[pl.ds(0, RB)], sem0).wait()
    pltpu.make_async_copy(buf1, out_hbm.at[pl.ds(0, RB)], sem1).wait()


def kernel(x, pos_table):
    mesh = plsc.VectorSubcoreMesh(core_axis_name="c", subcore_axis_name="s")
    k = pl.kernel(
        _sc_body,
        mesh=mesh,

        out_type=jax.ShapeDtypeStruct((BATCH, SEQLEN, EMBED), jnp.float32),
        scratch_types=[
            pltpu.VMEM((ROWS_PER_W, SEQLEN), jnp.float32),
            pltpu.VMEM((SEQLEN, EMBED), jnp.float32),
            pltpu.VMEM((RB, SEQLEN, EMBED), jnp.float32),
            pltpu.VMEM((RB, SEQLEN, EMBED), jnp.float32),
            pltpu.SemaphoreType.DMA,
            pltpu.SemaphoreType.DMA,
        ],
    )
    return k(x, pos_table)


# speedup vs baseline: 5.6025x; 5.6025x over previous
import jax
import jax.numpy as jnp
from jax import lax
from jax.experimental import pallas as pl
from jax.experimental.pallas import tpu as pltpu
from jax.experimental.pallas import tpu_sc as plsc

BATCH = 4096
SEQLEN = 200
EMBED = 64

NC = 2
NS = 16
NW = NC * NS
DT = EMBED // 8
UNITS = SEQLEN * DT
UNITS_PER_W = UNITS // NW
UNIT = 8 * BATCH
XROWS = 8


def _sc_body(xt_hbm, pos_hbm, out_hbm, xt_v, pos_v, buf0, buf1, sem0, sem1):
    wid = lax.axis_index("s") * NC + lax.axis_index("c")
    u_base = wid * UNITS_PER_W
    l_base = lax.min(u_base // DT, SEQLEN - XROWS)

    pltpu.sync_copy(xt_hbm.at[pl.ds(l_base, XROWS)], xt_v)
    pltpu.sync_copy(pos_hbm.at[pl.ds(l_base, XROWS)], pos_v)

    def compute_unit(buf, u):
        l_loc = u // DT - l_base
        dt = u % DT
        g = pl.multiple_of((dt // 2) * 16, 16)
        pv = pos_v[l_loc, pl.ds(g, 16)]
        lo = (dt % 2) == 0
        splats = [jnp.where(lo, pv[dd], pv[8 + dd]) for dd in range(8)]

        def bt_body(bth, carry):
            for h in range(2):
                bt = bth * 2 + h
                boff = pl.multiple_of(bt * 128, 128)
                for c in range(8):
                    xv = xt_v[l_loc, pl.ds(boff + c * 16, 16)]
                    for dd in range(8):
                        buf[pl.ds(boff * 8 + dd * 128 + c * 16, 16)] = (
                            xv + splats[dd]
                        )
            return carry

        lax.fori_loop(0, 16, bt_body, 0)

    def body(i, carry):
        u0 = u_base + 2 * i

        @pl.when(i > 0)
        def _():
            pltpu.make_async_copy(buf0, out_hbm.at[pl.ds(0, UNIT)], sem0).wait()

        compute_unit(buf0, u0)
        pltpu.make_async_copy(buf0, out_hbm.at[pl.ds(u0 * UNIT, UNIT)], sem0).start()

        @pl.when(i > 0)
        def _():
            pltpu.make_async_copy(buf1, out_hbm.at[pl.ds(0, UNIT)], sem1).wait()

        compute_unit(buf1, u0 + 1)
        pltpu.make_async_copy(
            buf1, out_hbm.at[pl.ds((u0 + 1) * UNIT, UNIT)], sem1
        ).start()
        return carry

    lax.fori_loop(0, UNITS_PER_W // 2, body, 0)
    pltpu.make_async_copy(buf0, out_hbm.at[pl.ds(0, UNIT)], sem0).wait()
    pltpu.make_async_copy(buf1, out_hbm.at[pl.ds(0, UNIT)], sem1).wait()


def kernel(x, pos_table):
    mesh = plsc.VectorSubcoreMesh(core_axis_name="c", subcore_axis_name="s")
    k = pl.kernel(
        _sc_body,
        mesh=mesh,
        compiler_params=pltpu.CompilerParams(use_tc_tiling_on_sc=False),
        out_type=jax.ShapeDtypeStruct((UNITS * UNIT,), jnp.float32),
        scratch_types=[
            pltpu.VMEM((XROWS, BATCH), jnp.float32),
            pltpu.VMEM((XROWS, EMBED), jnp.float32),
            pltpu.VMEM((UNIT,), jnp.float32),
            pltpu.VMEM((UNIT,), jnp.float32),
            pltpu.SemaphoreType.DMA,
            pltpu.SemaphoreType.DMA,
        ],
    )
    out_flat = k(x.T, pos_table)
    t = out_flat.reshape(SEQLEN, DT, BATCH // 128, 8, 128)
    return t.transpose(2, 4, 0, 1, 3).reshape(BATCH, SEQLEN, EMBED)
